# Initial kernel scaffold; baseline (speedup 1.0000x reference)
import functools

import jax
import jax.numpy as jnp
from jax import lax
from jax.experimental import pallas as pl
from jax.experimental.pallas import tpu as pltpu
from jax.experimental.pallas import tpu_sc as plsc

F32 = jnp.float32
D = 128
H = 256
E = 160000
N = 10000
G = 128
NC = 2
NS = 16
NW = NC * NS
E_PAD = 163840
N_PAD = 10016
OBJ_PAD = 12288


def _mesh():
    return plsc.VectorSubcoreMesh(core_axis_name="c", subcore_axis_name="s")


def _sc_gather(table, idx):
    n = idx.shape[0]
    nbw = n // (G * NW)

    @functools.partial(
        pl.kernel,
        out_type=jax.ShapeDtypeStruct((n, D), F32),
        mesh=_mesh(),
        scratch_types=[
            pltpu.VMEM((G,), jnp.int32),
            pltpu.VMEM((G, D), F32),
            pltpu.SemaphoreType.DMA,
        ],
    )
    def k(table_hbm, idx_hbm, out_hbm, idx_v, rows_v, sem):
        wid = lax.axis_index("s") * NC + lax.axis_index("c")

        def body(i, carry):
            base = (wid * nbw + i) * G
            pltpu.sync_copy(idx_hbm.at[pl.ds(base, G)], idx_v)
            pltpu.async_copy(table_hbm.at[idx_v], rows_v, sem).wait()
            pltpu.sync_copy(rows_v, out_hbm.at[pl.ds(base, G)])
            return carry

        lax.fori_loop(0, nbw, body, 0)

    return k(table, idx)


def _sc_scatter(ns_lo, ns_hi, no_lo, no_hi, sidx, oidx, zeros_np):
    nblk = E_PAD // G
    per_sub = nblk // NS
    rw = N_PAD // NS

    @functools.partial(
        pl.kernel,
        out_type=(jax.ShapeDtypeStruct((N_PAD, D), F32),
                  jax.ShapeDtypeStruct((N_PAD, D), F32)),
        mesh=_mesh(),
        scratch_types=[
            pltpu.VMEM((G,), jnp.int32),
            pltpu.VMEM((G, D), F32),
            pltpu.VMEM_SHARED((N_PAD, D), F32),
        ],
    )
    def k(nslo_h, nshi_h, nolo_h, nohi_h, sidx_h, oidx_h, zeros_h,
          plo_h, phi_h, idx_v, rows_v, acc):
        cid = lax.axis_index("c")
        sid = lax.axis_index("s")
        pltpu.sync_copy(zeros_h.at[pl.ds(sid * rw, rw)],
                        acc.at[pl.ds(sid * rw, rw)])
        plsc.subcore_barrier()

        def half(ns_h, no_h):
            def body(i, carry):
                base = (i * NS + sid) * G
                pltpu.sync_copy(sidx_h.at[pl.ds(base, G)], idx_v)
                pltpu.sync_copy(ns_h.at[pl.ds(base, G)], rows_v)
                pltpu.sync_copy(rows_v, acc.at[idx_v], add=True)
                pltpu.sync_copy(oidx_h.at[pl.ds(base, G)], idx_v)
                pltpu.sync_copy(no_h.at[pl.ds(base, G)], rows_v)
                pltpu.sync_copy(rows_v, acc.at[idx_v], add=True)
                return carry
            lax.fori_loop(0, per_sub, body, 0)

        @pl.when(cid == 0)
        def _():
            half(nslo_h, nolo_h)

        @pl.when(cid == 1)
        def _():
            half(nshi_h, nohi_h)

        plsc.subcore_barrier()

        @pl.when(cid == 0)
        def _():
            pltpu.sync_copy(acc.at[pl.ds(sid * rw, rw)],
                            plo_h.at[pl.ds(sid * rw, rw)])

        @pl.when(cid == 1)
        def _():
            pltpu.sync_copy(acc.at[pl.ds(sid * rw, rw)],
                            phi_h.at[pl.ds(sid * rw, rw)])

    return k(ns_lo, ns_hi, no_lo, no_hi, sidx, oidx, zeros_np)


def _sc_counts(sidx, oidx, ones_blk, zeros_np):
    nblk = E_PAD // G
    per_w = nblk // NW
    rw = N_PAD // NS

    @functools.partial(
        pl.kernel,
        out_type=(jax.ShapeDtypeStruct((N_PAD, D), F32),
                  jax.ShapeDtypeStruct((N_PAD, D), F32)),
        mesh=_mesh(),
        scratch_types=[
            pltpu.VMEM((G,), jnp.int32),
            pltpu.VMEM((G, D), F32),
            pltpu.VMEM_SHARED((N_PAD, D), F32),
        ],
    )
    def k(sidx_h, oidx_h, ones_h, zeros_h, c0_h, c1_h, idx_v, rows_v, acc):
        cid = lax.axis_index("c")
        sid = lax.axis_index("s")
        wid = sid * NC + cid
        pltpu.sync_copy(ones_h, rows_v)
        pltpu.sync_copy(zeros_h.at[pl.ds(sid * rw, rw)],
                        acc.at[pl.ds(sid * rw, rw)])
        plsc.subcore_barrier()

        def body(i, carry):
            base = (i * NW + wid) * G
            pltpu.sync_copy(sidx_h.at[pl.ds(base, G)], idx_v)
            pltpu.sync_copy(rows_v, acc.at[idx_v], add=True)
            pltpu.sync_copy(oidx_h.at[pl.ds(base, G)], idx_v)
            pltpu.sync_copy(rows_v, acc.at[idx_v], add=True)
            return carry

        lax.fori_loop(0, per_w, body, 0)
        plsc.subcore_barrier()

        @pl.when(cid == 0)
        def _():
            pltpu.sync_copy(acc.at[pl.ds(sid * rw, rw)],
                            c0_h.at[pl.ds(sid * rw, rw)])

        @pl.when(cid == 1)
        def _():
            pltpu.sync_copy(acc.at[pl.ds(sid * rw, rw)],
                            c1_h.at[pl.ds(sid * rw, rw)])

    return k(sidx, oidx, ones_blk, zeros_np)


def _tc_edge(cs, co, pv, w1, b1, w2, b2):
    R = 1024
    nb = E_PAD // R

    def body(cs_ref, co_ref, pv_ref, w1_ref, b1_ref, w2_ref, b2_ref,
             nslo_ref, nshi_ref, np_ref, nolo_ref, nohi_ref):
        h = (jnp.dot(cs_ref[...], w1_ref[0:D], preferred_element_type=F32)
             + jnp.dot(pv_ref[...], w1_ref[D:2 * D], preferred_element_type=F32)
             + jnp.dot(co_ref[...], w1_ref[2 * D:3 * D], preferred_element_type=F32)
             + b1_ref[...])
        h = jnp.maximum(h, 0.0)
        t = jnp.dot(h, w2_ref[...], preferred_element_type=F32) + b2_ref[...]
        t = jnp.maximum(t, 0.0)
        nslo_ref[...] = t[:, 0:D]
        nshi_ref[...] = t[:, D:2 * D]
        np_ref[...] = t[:, 2 * D:3 * D]
        nolo_ref[...] = t[:, 3 * D:4 * D]
        nohi_ref[...] = t[:, 4 * D:5 * D]

    row_spec = pl.BlockSpec((R, D), lambda i: (i, 0))
    full = lambda shape: pl.BlockSpec(shape, lambda i: (0, 0))
    return pl.pallas_call(
        body,
        grid=(nb,),
        in_specs=[row_spec, row_spec, row_spec,
                  full((3 * D, H)), full((1, H)),
                  full((H, 5 * D)), full((1, 5 * D))],
        out_specs=[row_spec] * 5,
        out_shape=[jax.ShapeDtypeStruct((E_PAD, D), F32)] * 5,
    )(cs, co, pv, w1, b1.reshape(1, -1), w2, b2.reshape(1, -1))


def _tc_node(plo, phi, c0, c1, w3, b3, w4, b4):
    R = 2504
    nb = N_PAD // R

    def body(plo_ref, phi_ref, c0_ref, c1_ref, w3_ref, b3_ref, w4_ref, b4_ref,
             out_ref):
        inv = 1.0 / jnp.maximum(c0_ref[...] + c1_ref[...], 1.0)
        a = plo_ref[...] * inv
        b = phi_ref[...] * inv
        h = (jnp.dot(a, w3_ref[0:D], preferred_element_type=F32)
             + jnp.dot(b, w3_ref[D:2 * D], preferred_element_type=F32)
             + b3_ref[...])
        h = jnp.maximum(h, 0.0)
        out = jnp.dot(h, w4_ref[...], preferred_element_type=F32) + b4_ref[...]
        out_ref[...] = jnp.maximum(out, 0.0)

    row_spec = pl.BlockSpec((R, D), lambda i: (i, 0))
    full = lambda shape: pl.BlockSpec(shape, lambda i: (0, 0))
    return pl.pallas_call(
        body,
        grid=(nb,),
        in_specs=[row_spec, row_spec, row_spec, row_spec,
                  full((2 * D, H)), full((1, H)),
                  full((H, D)), full((1, D))],
        out_specs=row_spec,
        out_shape=jax.ShapeDtypeStruct((N_PAD, D), F32),
    )(plo, phi, c0, c1, w3, b3.reshape(1, -1), w4, b4.reshape(1, -1))


def kernel(objs, triples, enc_text_feat, enc_rel_feat, obj_emb_ec,
           pred_emb_ec, gconv_params):
    del enc_text_feat, enc_rel_feat
    i32 = jnp.int32
    s_idx = triples[:, 0].astype(i32)
    p_idx = triples[:, 1].astype(i32)
    o_idx = triples[:, 2].astype(i32)

    pad_e = E_PAD - E
    s_pad = jnp.concatenate([s_idx, jnp.full((pad_e,), N, i32)])
    o_pad = jnp.concatenate([o_idx, jnp.full((pad_e,), N, i32)])
    p_pad = jnp.concatenate([p_idx, jnp.zeros((pad_e,), i32)])
    objs_pad = jnp.concatenate([objs.astype(i32),
                                jnp.zeros((OBJ_PAD - N,), i32)])

    obj_table = jnp.pad(obj_emb_ec.astype(F32), ((0, 3), (0, 0)))
    zeros_np = jnp.zeros((N_PAD, D), F32)
    ones_blk = jnp.ones((G, D), F32)

    obj_g = _sc_gather(obj_table, objs_pad)
    obj_embed = obj_g[:N]
    pred_g = _sc_gather(pred_emb_ec.astype(F32), p_pad)
    pred_embed = pred_g[:E]

    c0, c1 = _sc_counts(s_pad, o_pad, ones_blk, zeros_np)

    obj_vecs_t = jnp.concatenate([obj_embed, jnp.zeros((N_PAD - N, D), F32)])
    pred_vecs = pred_g
    for lp in gconv_params:
        cs = _sc_gather(obj_vecs_t, s_pad)
        co = _sc_gather(obj_vecs_t, o_pad)
        n1, n2 = lp['net1'], lp['net2']
        nslo, nshi, npred, nolo, nohi = _tc_edge(
            cs, co, pred_vecs, n1['W1'], n1['b1'], n1['W2'], n1['b2'])
        plo, phi = _sc_scatter(nslo, nshi, nolo, nohi, s_pad, o_pad, zeros_np)
        obj_vecs_t = _tc_node(plo, phi, c0, c1,
                              n2['W1'], n2['b1'], n2['W2'], n2['b2'])
        pred_vecs = npred

    return obj_embed, pred_embed, obj_vecs_t[:N], pred_vecs[:E]


# speedup vs baseline: 1.7547x; 1.7547x over previous
import functools

import jax
import jax.numpy as jnp
from jax import lax
from jax.experimental import pallas as pl
from jax.experimental.pallas import tpu as pltpu
from jax.experimental.pallas import tpu_sc as plsc

F32 = jnp.float32
D = 128
H = 256
E = 160000
N = 10000
G = 128
NC = 2
NS = 16
NW = NC * NS
E_PAD = 163840
N_PAD = 10112
OBJ_PAD = 12288


def _mesh():
    return plsc.VectorSubcoreMesh(core_axis_name="c", subcore_axis_name="s")


def _sc_gather(table, idx):
    n = idx.shape[0]
    nbw = n // (G * NW)

    @functools.partial(
        pl.kernel,
        out_type=jax.ShapeDtypeStruct((n, D), F32),
        mesh=_mesh(),
        scratch_types=[
            pltpu.VMEM((G,), jnp.int32),
            pltpu.VMEM((G, D), F32),
            pltpu.SemaphoreType.DMA,
        ],
    )
    def k(table_hbm, idx_hbm, out_hbm, idx_v, rows_v, sem):
        wid = lax.axis_index("s") * NC + lax.axis_index("c")

        def body(i, carry):
            base = (wid * nbw + i) * G
            pltpu.sync_copy(idx_hbm.at[pl.ds(base, G)], idx_v)
            pltpu.async_copy(table_hbm.at[idx_v], rows_v, sem).wait()
            pltpu.sync_copy(rows_v, out_hbm.at[pl.ds(base, G)])
            return carry

        lax.fori_loop(0, nbw, body, 0)

    return k(table, idx)


def _sc_scatter(ns_lo, ns_hi, no_lo, no_hi, sidx, oidx, zeros_np):
    nblk = E_PAD // G
    per_sub = nblk // NS
    rw = N_PAD // NS

    @functools.partial(
        pl.kernel,
        out_type=(jax.ShapeDtypeStruct((N_PAD, D), F32),
                  jax.ShapeDtypeStruct((N_PAD, D), F32)),
        mesh=_mesh(),
        scratch_types=[
            pltpu.VMEM((G,), jnp.int32),
            pltpu.VMEM((G, D), F32),
            pltpu.VMEM_SHARED((N_PAD, D), F32),
        ],
    )
    def k(nslo_h, nshi_h, nolo_h, nohi_h, sidx_h, oidx_h, zeros_h,
          plo_h, phi_h, idx_v, rows_v, acc):
        cid = lax.axis_index("c")
        sid = lax.axis_index("s")
        pltpu.sync_copy(zeros_h.at[pl.ds(sid * rw, rw)],
                        acc.at[pl.ds(sid * rw, rw)])
        plsc.subcore_barrier()

        def half(ns_h, no_h):
            def body(i, carry):
                base = (i * NS + sid) * G
                pltpu.sync_copy(sidx_h.at[pl.ds(base, G)], idx_v)
                pltpu.sync_copy(ns_h.at[pl.ds(base, G)], rows_v)
                pltpu.sync_copy(rows_v, acc.at[idx_v], add=True)
                pltpu.sync_copy(oidx_h.at[pl.ds(base, G)], idx_v)
                pltpu.sync_copy(no_h.at[pl.ds(base, G)], rows_v)
                pltpu.sync_copy(rows_v, acc.at[idx_v], add=True)
                return carry
            lax.fori_loop(0, per_sub, body, 0)

        @pl.when(cid == 0)
        def _():
            half(nslo_h, nolo_h)

        @pl.when(cid == 1)
        def _():
            half(nshi_h, nohi_h)

        plsc.subcore_barrier()

        @pl.when(cid == 0)
        def _():
            pltpu.sync_copy(acc.at[pl.ds(sid * rw, rw)],
                            plo_h.at[pl.ds(sid * rw, rw)])

        @pl.when(cid == 1)
        def _():
            pltpu.sync_copy(acc.at[pl.ds(sid * rw, rw)],
                            phi_h.at[pl.ds(sid * rw, rw)])

    return k(ns_lo, ns_hi, no_lo, no_hi, sidx, oidx, zeros_np)


def _sc_counts(sidx, oidx, ones_blk, zeros_np):
    nblk = E_PAD // G
    per_w = nblk // NW
    rw = N_PAD // NS

    @functools.partial(
        pl.kernel,
        out_type=(jax.ShapeDtypeStruct((N_PAD, D), F32),
                  jax.ShapeDtypeStruct((N_PAD, D), F32)),
        mesh=_mesh(),
        scratch_types=[
            pltpu.VMEM((G,), jnp.int32),
            pltpu.VMEM((G, D), F32),
            pltpu.VMEM_SHARED((N_PAD, D), F32),
        ],
    )
    def k(sidx_h, oidx_h, ones_h, zeros_h, c0_h, c1_h, idx_v, rows_v, acc):
        cid = lax.axis_index("c")
        sid = lax.axis_index("s")
        wid = sid * NC + cid
        pltpu.sync_copy(ones_h, rows_v)
        pltpu.sync_copy(zeros_h.at[pl.ds(sid * rw, rw)],
                        acc.at[pl.ds(sid * rw, rw)])
        plsc.subcore_barrier()

        def body(i, carry):
            base = (i * NW + wid) * G
            pltpu.sync_copy(sidx_h.at[pl.ds(base, G)], idx_v)
            pltpu.sync_copy(rows_v, acc.at[idx_v], add=True)
            pltpu.sync_copy(oidx_h.at[pl.ds(base, G)], idx_v)
            pltpu.sync_copy(rows_v, acc.at[idx_v], add=True)
            return carry

        lax.fori_loop(0, per_w, body, 0)
        plsc.subcore_barrier()

        @pl.when(cid == 0)
        def _():
            pltpu.sync_copy(acc.at[pl.ds(sid * rw, rw)],
                            c0_h.at[pl.ds(sid * rw, rw)])

        @pl.when(cid == 1)
        def _():
            pltpu.sync_copy(acc.at[pl.ds(sid * rw, rw)],
                            c1_h.at[pl.ds(sid * rw, rw)])

    return k(sidx, oidx, ones_blk, zeros_np)


def _tc_edge(cs, co, pv, w1, b1, w2, b2):
    R = 1024
    nb = E_PAD // R

    def body(cs_ref, co_ref, pv_ref, w1_ref, b1_ref, w2_ref, b2_ref,
             nslo_ref, nshi_ref, np_ref, nolo_ref, nohi_ref):
        h = (jnp.dot(cs_ref[...], w1_ref[0:D], preferred_element_type=F32)
             + jnp.dot(pv_ref[...], w1_ref[D:2 * D], preferred_element_type=F32)
             + jnp.dot(co_ref[...], w1_ref[2 * D:3 * D], preferred_element_type=F32)
             + b1_ref[...])
        h = jnp.maximum(h, 0.0)
        t = jnp.dot(h, w2_ref[...], preferred_element_type=F32) + b2_ref[...]
        t = jnp.maximum(t, 0.0)
        nslo_ref[...] = t[:, 0:D]
        nshi_ref[...] = t[:, D:2 * D]
        np_ref[...] = t[:, 2 * D:3 * D]
        nolo_ref[...] = t[:, 3 * D:4 * D]
        nohi_ref[...] = t[:, 4 * D:5 * D]

    row_spec = pl.BlockSpec((R, D), lambda i: (i, 0))
    full = lambda shape: pl.BlockSpec(shape, lambda i: (0, 0))
    return pl.pallas_call(
        body,
        grid=(nb,),
        in_specs=[row_spec, row_spec, row_spec,
                  full((3 * D, H)), full((1, H)),
                  full((H, 5 * D)), full((1, 5 * D))],
        out_specs=[row_spec] * 5,
        out_shape=[jax.ShapeDtypeStruct((E_PAD, D), F32)] * 5,
    )(cs, co, pv, w1, b1.reshape(1, -1), w2, b2.reshape(1, -1))


def _tc_node(plo, phi, c0, c1, w3, b3, w4, b4):
    R = 2528
    nb = N_PAD // R

    def body(plo_ref, phi_ref, c0_ref, c1_ref, w3_ref, b3_ref, w4_ref, b4_ref,
             out_ref):
        inv = 1.0 / jnp.maximum(c0_ref[...] + c1_ref[...], 1.0)
        a = plo_ref[...] * inv
        b = phi_ref[...] * inv
        h = (jnp.dot(a, w3_ref[0:D], preferred_element_type=F32)
             + jnp.dot(b, w3_ref[D:2 * D], preferred_element_type=F32)
             + b3_ref[...])
        h = jnp.maximum(h, 0.0)
        out = jnp.dot(h, w4_ref[...], preferred_element_type=F32) + b4_ref[...]
        out_ref[...] = jnp.maximum(out, 0.0)

    row_spec = pl.BlockSpec((R, D), lambda i: (i, 0))
    full = lambda shape: pl.BlockSpec(shape, lambda i: (0, 0))
    return pl.pallas_call(
        body,
        grid=(nb,),
        in_specs=[row_spec, row_spec, row_spec, row_spec,
                  full((2 * D, H)), full((1, H)),
                  full((H, D)), full((1, D))],
        out_specs=row_spec,
        out_shape=jax.ShapeDtypeStruct((N_PAD, D), F32),
    )(plo, phi, c0, c1, w3, b3.reshape(1, -1), w4, b4.reshape(1, -1))


def kernel(objs, triples, enc_text_feat, enc_rel_feat, obj_emb_ec,
           pred_emb_ec, gconv_params):
    del enc_text_feat, enc_rel_feat
    i32 = jnp.int32
    s_idx = triples[:, 0].astype(i32)
    p_idx = triples[:, 1].astype(i32)
    o_idx = triples[:, 2].astype(i32)

    pad_e = E_PAD - E
    s_pad = jnp.concatenate([s_idx, jnp.full((pad_e,), N, i32)])
    o_pad = jnp.concatenate([o_idx, jnp.full((pad_e,), N, i32)])
    p_pad = jnp.concatenate([p_idx, jnp.zeros((pad_e,), i32)])
    objs_pad = jnp.concatenate([objs.astype(i32),
                                jnp.zeros((OBJ_PAD - N,), i32)])

    obj_table = jnp.pad(obj_emb_ec.astype(F32), ((0, 3), (0, 0)))
    zeros_np = jnp.zeros((N_PAD, D), F32)
    ones_blk = jnp.ones((G, D), F32)

    obj_g = _sc_gather(obj_table, objs_pad)
    obj_embed = obj_g[:N]
    pred_g = _sc_gather(pred_emb_ec.astype(F32), p_pad)
    pred_embed = pred_g[:E]

    c0, c1 = _sc_counts(s_pad, o_pad, ones_blk, zeros_np)

    obj_vecs_t = jnp.concatenate([obj_embed, jnp.zeros((N_PAD - N, D), F32)])
    pred_vecs = pred_g
    for lp in gconv_params:
        cs = _sc_gather(obj_vecs_t, s_pad)
        co = _sc_gather(obj_vecs_t, o_pad)
        n1, n2 = lp['net1'], lp['net2']
        nslo, nshi, npred, nolo, nohi = _tc_edge(
            cs, co, pred_vecs, n1['W1'], n1['b1'], n1['W2'], n1['b2'])
        plo, phi = _sc_scatter(nslo, nshi, nolo, nohi, s_pad, o_pad, zeros_np)
        obj_vecs_t = _tc_node(plo, phi, c0, c1,
                              n2['W1'], n2['b1'], n2['W2'], n2['b2'])
        pred_vecs = npred

    return obj_embed, pred_embed, obj_vecs_t[:N], pred_vecs[:E]


# speedup vs baseline: 2.0863x; 1.1890x over previous
import functools

import jax
import jax.numpy as jnp
from jax import lax
from jax.experimental import pallas as pl
from jax.experimental.pallas import tpu as pltpu
from jax.experimental.pallas import tpu_sc as plsc

F32 = jnp.float32
D = 128
H = 256
E = 160000
N = 10000
G = 128
NC = 2
NS = 16
NW = NC * NS
E_PAD = 163840
N_PAD = 10112
OBJ_PAD = 16384
NB = 4
ROWB = G * D * 4


def _mesh():
    return plsc.VectorSubcoreMesh(core_axis_name="c", subcore_axis_name="s")


def _sc_gather(table, idx):
    n = idx.shape[0]
    nbw = n // (G * NW)
    ngrp = nbw // NB

    @functools.partial(
        pl.kernel,
        out_type=jax.ShapeDtypeStruct((n, D), F32),
        mesh=_mesh(),
        scratch_types=[
            pltpu.VMEM((nbw * G,), jnp.int32),
            pltpu.VMEM((NB, G, D), F32),
            pltpu.SemaphoreType.DMA((NB,)),
            pltpu.SemaphoreType.DMA((NB,)),
        ],
    )
    def k(table_hbm, idx_hbm, out_hbm, idx_v, rows_v, gsem, wsem):
        wid = lax.axis_index("s") * NC + lax.axis_index("c")
        wbase = wid * nbw * G
        pltpu.sync_copy(idx_hbm.at[pl.ds(wbase, nbw * G)], idx_v)

        def fire(i, b):
            pltpu.async_copy(table_hbm.at[idx_v.at[pl.ds(i * G, G)]],
                             rows_v.at[b], gsem.at[b])

        for b in range(NB):
            fire(b, b)

        def grp(g, carry):
            i0 = g * NB
            for b in range(NB):
                pltpu.make_async_copy(out_hbm.at[pl.ds(wbase, G)],
                                      rows_v.at[b], gsem.at[b]).wait()
                pltpu.async_copy(rows_v.at[b],
                                 out_hbm.at[pl.ds(wbase + (i0 + b) * G, G)],
                                 wsem.at[b])
            for b in range(NB):
                pltpu.make_async_copy(out_hbm.at[pl.ds(wbase, G)],
                                      rows_v.at[b], wsem.at[b]).wait()

                @pl.when(g + 1 < ngrp)
                def _():
                    fire(i0 + NB + b, b)
            return carry

        lax.fori_loop(0, ngrp, grp, 0)

    return k(table, idx)


def _sc_scatter(ns_lo, ns_hi, no_lo, no_hi, sidx2, oidx2, zeros_np):
    nblk = E_PAD // G
    per_sub = nblk // NS
    NBS = 2
    ngrp = per_sub // NBS
    rw = N_PAD // NS

    @functools.partial(
        pl.kernel,
        out_type=(jax.ShapeDtypeStruct((N_PAD, D), F32),
                  jax.ShapeDtypeStruct((N_PAD, D), F32)),
        mesh=_mesh(),
        scratch_types=[
            pltpu.VMEM((per_sub, G), jnp.int32),
            pltpu.VMEM((NBS, G, D), F32),
            pltpu.SemaphoreType.DMA((NBS,)),
            pltpu.SemaphoreType.DMA((NBS,)),
            pltpu.VMEM_SHARED((N_PAD, D), F32),
        ],
    )
    def k(nslo_h, nshi_h, nolo_h, nohi_h, sidx_h, oidx_h, zeros_h,
          plo_h, phi_h, idx_v, rows_v, lsem, ssem, acc):
        cid = lax.axis_index("c")
        sid = lax.axis_index("s")
        pltpu.sync_copy(zeros_h.at[pl.ds(sid * rw, rw)],
                        acc.at[pl.ds(sid * rw, rw)])
        plsc.subcore_barrier()

        def half(data_h, idx_h):
            pltpu.sync_copy(idx_h.at[pl.ds(sid * per_sub, per_sub)], idx_v)

            def fire(i, b):
                base = (sid * per_sub + i) * G
                pltpu.async_copy(data_h.at[pl.ds(base, G)],
                                 rows_v.at[b], lsem.at[b])

            for b in range(NBS):
                fire(b, b)

            def grp(g, carry):
                i0 = g * NBS
                for b in range(NBS):
                    pltpu.make_async_copy(data_h.at[pl.ds(0, G)],
                                          rows_v.at[b], lsem.at[b]).wait()
                    pltpu.async_copy(rows_v.at[b], acc.at[idx_v.at[i0 + b]],
                                     ssem.at[b], add=True)
                for b in range(NBS):
                    pltpu.make_async_copy(data_h.at[pl.ds(0, G)],
                                          rows_v.at[b], ssem.at[b]).wait()

                    @pl.when(g + 1 < ngrp)
                    def _():
                        fire(i0 + NBS + b, b)
                return carry

            lax.fori_loop(0, ngrp, grp, 0)

        @pl.when(cid == 0)
        def _():
            half(nslo_h, sidx_h)
            half(nolo_h, oidx_h)

        @pl.when(cid == 1)
        def _():
            half(nshi_h, sidx_h)
            half(nohi_h, oidx_h)

        plsc.subcore_barrier()

        @pl.when(cid == 0)
        def _():
            pltpu.sync_copy(acc.at[pl.ds(sid * rw, rw)],
                            plo_h.at[pl.ds(sid * rw, rw)])

        @pl.when(cid == 1)
        def _():
            pltpu.sync_copy(acc.at[pl.ds(sid * rw, rw)],
                            phi_h.at[pl.ds(sid * rw, rw)])

    return k(ns_lo, ns_hi, no_lo, no_hi, sidx2, oidx2, zeros_np)


def _sc_counts(sidx2, oidx2, ones_blk, zeros_np):
    nblk = E_PAD // G
    per_w = nblk // NW
    rw = N_PAD // NS

    @functools.partial(
        pl.kernel,
        out_type=(jax.ShapeDtypeStruct((N_PAD, D), F32),
                  jax.ShapeDtypeStruct((N_PAD, D), F32)),
        mesh=_mesh(),
        scratch_types=[
            pltpu.VMEM((per_w, G), jnp.int32),
            pltpu.VMEM((per_w, G), jnp.int32),
            pltpu.VMEM((G, D), F32),
            pltpu.SemaphoreType.DMA,
            pltpu.VMEM_SHARED((N_PAD, D), F32),
        ],
    )
    def k(sidx_h, oidx_h, ones_h, zeros_h, c0_h, c1_h,
          sidx_v, oidx_v, rows_v, ssem, acc):
        cid = lax.axis_index("c")
        sid = lax.axis_index("s")
        wid = sid * NC + cid
        pltpu.sync_copy(ones_h, rows_v)
        pltpu.sync_copy(sidx_h.at[pl.ds(wid * per_w, per_w)], sidx_v)
        pltpu.sync_copy(oidx_h.at[pl.ds(wid * per_w, per_w)], oidx_v)
        pltpu.sync_copy(zeros_h.at[pl.ds(sid * rw, rw)],
                        acc.at[pl.ds(sid * rw, rw)])
        plsc.subcore_barrier()

        def body(i, carry):
            pltpu.async_copy(rows_v, acc.at[sidx_v.at[i]], ssem, add=True)
            pltpu.async_copy(rows_v, acc.at[oidx_v.at[i]], ssem, add=True)

            @pl.when(i >= 1)
            def _():
                pltpu.make_async_copy(zeros_h.at[pl.ds(0, G)], rows_v, ssem).wait()
                pltpu.make_async_copy(zeros_h.at[pl.ds(0, G)], rows_v, ssem).wait()
            return carry

        lax.fori_loop(0, per_w, body, 0)
        pltpu.make_async_copy(zeros_h.at[pl.ds(0, G)], rows_v, ssem).wait()
        pltpu.make_async_copy(zeros_h.at[pl.ds(0, G)], rows_v, ssem).wait()
        plsc.subcore_barrier()

        @pl.when(cid == 0)
        def _():
            pltpu.sync_copy(acc.at[pl.ds(sid * rw, rw)],
                            c0_h.at[pl.ds(sid * rw, rw)])

        @pl.when(cid == 1)
        def _():
            pltpu.sync_copy(acc.at[pl.ds(sid * rw, rw)],
                            c1_h.at[pl.ds(sid * rw, rw)])

    return k(sidx2, oidx2, ones_blk, zeros_np)


def _tc_edge(cs, co, pv, w1, b1, w2, b2):
    R = 1024
    nb = E_PAD // R

    def body(cs_ref, co_ref, pv_ref, w1_ref, b1_ref, w2_ref, b2_ref,
             nslo_ref, nshi_ref, np_ref, nolo_ref, nohi_ref):
        h = (jnp.dot(cs_ref[...], w1_ref[0:D], preferred_element_type=F32)
             + jnp.dot(pv_ref[...], w1_ref[D:2 * D], preferred_element_type=F32)
             + jnp.dot(co_ref[...], w1_ref[2 * D:3 * D], preferred_element_type=F32)
             + b1_ref[...])
        h = jnp.maximum(h, 0.0)
        t = jnp.dot(h, w2_ref[...], preferred_element_type=F32) + b2_ref[...]
        t = jnp.maximum(t, 0.0)
        nslo_ref[...] = t[:, 0:D]
        nshi_ref[...] = t[:, D:2 * D]
        np_ref[...] = t[:, 2 * D:3 * D]
        nolo_ref[...] = t[:, 3 * D:4 * D]
        nohi_ref[...] = t[:, 4 * D:5 * D]

    row_spec = pl.BlockSpec((R, D), lambda i: (i, 0))
    full = lambda shape: pl.BlockSpec(shape, lambda i: (0, 0))
    return pl.pallas_call(
        body,
        grid=(nb,),
        in_specs=[row_spec, row_spec, row_spec,
                  full((3 * D, H)), full((1, H)),
                  full((H, 5 * D)), full((1, 5 * D))],
        out_specs=[row_spec] * 5,
        out_shape=[jax.ShapeDtypeStruct((E_PAD, D), F32)] * 5,
    )(cs, co, pv, w1, b1.reshape(1, -1), w2, b2.reshape(1, -1))


def _tc_node(plo, phi, c0, c1, w3, b3, w4, b4):
    R = 2528
    nb = N_PAD // R

    def body(plo_ref, phi_ref, c0_ref, c1_ref, w3_ref, b3_ref, w4_ref, b4_ref,
             out_ref):
        inv = 1.0 / jnp.maximum(c0_ref[...] + c1_ref[...], 1.0)
        a = plo_ref[...] * inv
        b = phi_ref[...] * inv
        h = (jnp.dot(a, w3_ref[0:D], preferred_element_type=F32)
             + jnp.dot(b, w3_ref[D:2 * D], preferred_element_type=F32)
             + b3_ref[...])
        h = jnp.maximum(h, 0.0)
        out = jnp.dot(h, w4_ref[...], preferred_element_type=F32) + b4_ref[...]
        out_ref[...] = jnp.maximum(out, 0.0)

    row_spec = pl.BlockSpec((R, D), lambda i: (i, 0))
    full = lambda shape: pl.BlockSpec(shape, lambda i: (0, 0))
    return pl.pallas_call(
        body,
        grid=(nb,),
        in_specs=[row_spec, row_spec, row_spec, row_spec,
                  full((2 * D, H)), full((1, H)),
                  full((H, D)), full((1, D))],
        out_specs=row_spec,
        out_shape=jax.ShapeDtypeStruct((N_PAD, D), F32),
    )(plo, phi, c0, c1, w3, b3.reshape(1, -1), w4, b4.reshape(1, -1))


def kernel(objs, triples, enc_text_feat, enc_rel_feat, obj_emb_ec,
           pred_emb_ec, gconv_params):
    del enc_text_feat, enc_rel_feat
    i32 = jnp.int32
    s_idx = triples[:, 0].astype(i32)
    p_idx = triples[:, 1].astype(i32)
    o_idx = triples[:, 2].astype(i32)

    pad_e = E_PAD - E
    s_pad = jnp.concatenate([s_idx, jnp.full((pad_e,), N, i32)])
    o_pad = jnp.concatenate([o_idx, jnp.full((pad_e,), N, i32)])
    p_pad = jnp.concatenate([p_idx, jnp.zeros((pad_e,), i32)])
    objs_pad = jnp.concatenate([objs.astype(i32),
                                jnp.zeros((OBJ_PAD - N,), i32)])
    sidx2 = s_pad.reshape(E_PAD // G, G)
    oidx2 = o_pad.reshape(E_PAD // G, G)

    obj_table = jnp.pad(obj_emb_ec.astype(F32), ((0, 3), (0, 0)))
    zeros_np = jnp.zeros((N_PAD, D), F32)
    ones_blk = jnp.ones((G, D), F32)

    obj_g = _sc_gather(obj_table, objs_pad)
    obj_embed = obj_g[:N]
    pred_g = _sc_gather(pred_emb_ec.astype(F32), p_pad)
    pred_embed = pred_g[:E]

    c0, c1 = _sc_counts(sidx2, oidx2, ones_blk, zeros_np)

    obj_vecs_t = jnp.concatenate([obj_embed, jnp.zeros((N_PAD - N, D), F32)])
    pred_vecs = pred_g
    for lp in gconv_params:
        cs = _sc_gather(obj_vecs_t, s_pad)
        co = _sc_gather(obj_vecs_t, o_pad)
        n1, n2 = lp['net1'], lp['net2']
        nslo, nshi, npred, nolo, nohi = _tc_edge(
            cs, co, pred_vecs, n1['W1'], n1['b1'], n1['W2'], n1['b2'])
        plo, phi = _sc_scatter(nslo, nshi, nolo, nohi, sidx2, oidx2, zeros_np)
        obj_vecs_t = _tc_node(plo, phi, c0, c1,
                              n2['W1'], n2['b1'], n2['W2'], n2['b2'])
        pred_vecs = npred

    return obj_embed, pred_embed, obj_vecs_t[:N], pred_vecs[:E]
